# Initial kernel scaffold; baseline (speedup 1.0000x reference)
import jax
import jax.numpy as jnp
from jax.experimental import pallas as pl


def kernel(input, nodes, clicked):
    raise NotImplementedError("write your pallas kernel here")


# speedup vs baseline: 34.7302x; 34.7302x over previous
import functools

import jax
import jax.numpy as jnp
from jax import lax
from jax.experimental import pallas as pl
from jax.experimental.pallas import tpu as pltpu
from jax.experimental.pallas import tpu_sc as plsc

_N_NODES = 100000
_N_EDGES = 3276800
_N_CLICKED = 64
_PI = 0.25

_LANE = 128
_ROWS = 784
_NPAD = _ROWS * _LANE
_NC, _NS = 2, 16
_NW = _NC * _NS
_ROWS_PER_W = _N_EDGES // _NW // _LANE
_SLICE = _NPAD // _NS

_sc_mesh = plsc.VectorSubcoreMesh(
    core_axis_name="c", subcore_axis_name="s", num_cores=_NC, num_subcores=_NS
)


@functools.partial(
    pl.kernel,
    out_type=jax.ShapeDtypeStruct((_NC, _NPAD), jnp.float32),
    mesh=_sc_mesh,
    scratch_types=[
        pltpu.VMEM((_ROWS_PER_W, _LANE), jnp.int32),
        pltpu.VMEM((_LANE,), jnp.float32),
        pltpu.VMEM_SHARED((_NPAD,), jnp.float32),
    ],
)
def _sc_count(nodes0_hbm, zeros_hbm, out_hbm, idx_v, ones_v, shared):
    cid = lax.axis_index("c")
    sid = lax.axis_index("s")

    pltpu.sync_copy(
        zeros_hbm.at[pl.ds(sid * _SLICE, _SLICE)],
        shared.at[pl.ds(sid * _SLICE, _SLICE)],
    )
    for k in range(_LANE // 16):
        ones_v[pl.ds(k * 16, 16)] = jnp.ones((16,), jnp.float32)

    wid = cid * _NS + sid
    pltpu.sync_copy(nodes0_hbm.at[pl.ds(wid * _ROWS_PER_W, _ROWS_PER_W)], idx_v)

    plsc.subcore_barrier()

    def body(j, carry):
        pltpu.sync_copy(ones_v, shared.at[idx_v.at[j]], add=True)
        return carry

    lax.fori_loop(0, _ROWS_PER_W, body, 0)

    plsc.subcore_barrier()

    pltpu.sync_copy(
        shared.at[pl.ds(sid * _SLICE, _SLICE)],
        out_hbm.at[cid, pl.ds(sid * _SLICE, _SLICE)],
    )


def _tc_loss_body(counts_ref, x_ref, clicked_ref, out_ref):
    c = counts_ref[0] + counts_ref[1]
    row = lax.broadcasted_iota(jnp.int32, (_ROWS, _LANE), 0)
    col = lax.broadcasted_iota(jnp.int32, (_ROWS, _LANE), 1)
    ids = row * _LANE + col
    x = x_ref[...]

    mult = jnp.zeros((_ROWS, _LANE), jnp.float32)
    for i in range(_N_CLICKED):
        mult = mult + jnp.where(ids == clicked_ref[i], 1.0, 0.0)

    valid = ids < _N_NODES
    present = (c > 0.0) & valid
    neg = present & (mult == 0.0)

    sp_p = jnp.maximum(x, 0.0) + jnp.log1p(jnp.exp(-jnp.abs(x)))
    sp_n = sp_p - x

    s_neg = jnp.sum(jnp.where(neg, sp_p, 0.0))
    n_neg = jnp.sum(jnp.where(neg, 1.0, 0.0))
    rp_minus = jnp.sum(mult * sp_p)
    rp_plus = jnp.sum(mult * sp_n)

    loss_p = _PI * rp_plus / _N_CLICKED
    loss_u = jnp.maximum(s_neg / n_neg - _PI * rp_minus / _N_CLICKED, 0.0)
    out_ref[0, 0] = loss_p + loss_u


_tc_loss = pl.pallas_call(
    _tc_loss_body,
    out_shape=jax.ShapeDtypeStruct((1, 1), jnp.float32),
    in_specs=[
        pl.BlockSpec(memory_space=pltpu.VMEM),
        pl.BlockSpec(memory_space=pltpu.VMEM),
        pl.BlockSpec(memory_space=pltpu.SMEM),
    ],
    out_specs=pl.BlockSpec(memory_space=pltpu.SMEM),
)


def kernel(input, nodes, clicked):
    nodes0 = nodes[0].reshape(_N_EDGES // _LANE, _LANE)
    zeros = jnp.zeros((_NPAD,), jnp.float32)
    counts = _sc_count(nodes0, zeros)
    x2 = jnp.pad(input, (0, _NPAD - _N_NODES)).reshape(_ROWS, _LANE)
    c3 = counts.reshape(_NC, _ROWS, _LANE)
    loss = _tc_loss(c3, x2, clicked)
    return loss[0, 0]


# speedup vs baseline: 47.1450x; 1.3575x over previous
import functools

import jax
import jax.numpy as jnp
from jax import lax
from jax.experimental import pallas as pl
from jax.experimental.pallas import tpu as pltpu
from jax.experimental.pallas import tpu_sc as plsc

_N_NODES = 100000
_N_EDGES = 3276800
_N_CLICKED = 64
_PI = 0.25

_LANE = 128
_ROWS = 784
_NPAD = _ROWS * _LANE
_NC, _NS = 2, 16
_NW = _NC * _NS
_ROWS_PER_W = _N_EDGES // _NW // _LANE
_SLICE = _NPAD // _NS

_sc_mesh = plsc.VectorSubcoreMesh(
    core_axis_name="c", subcore_axis_name="s", num_cores=_NC, num_subcores=_NS
)


_N_CHUNKS = 5
_CHUNK = _N_EDGES // _NW // _N_CHUNKS


@functools.partial(
    pl.kernel,
    out_type=jax.ShapeDtypeStruct((_NC, _NPAD), jnp.float32),
    mesh=_sc_mesh,
    scratch_types=[
        pltpu.VMEM((_N_CHUNKS, _CHUNK), jnp.int32),
        pltpu.VMEM((_CHUNK,), jnp.float32),
        pltpu.VMEM_SHARED((_NPAD,), jnp.float32),
    ],
    compiler_params=pltpu.CompilerParams(use_tc_tiling_on_sc=False),
)
def _sc_count(nodes3_hbm, zeros_hbm, ones_hbm, out_hbm, idx_v, ones_v, shared):
    cid = lax.axis_index("c")
    sid = lax.axis_index("s")

    pltpu.sync_copy(
        zeros_hbm.at[pl.ds(sid * _SLICE, _SLICE)],
        shared.at[pl.ds(sid * _SLICE, _SLICE)],
    )
    pltpu.sync_copy(ones_hbm, ones_v)
    wid = cid * _NS + sid
    pltpu.sync_copy(nodes3_hbm.at[0, wid], idx_v)

    plsc.subcore_barrier()

    for j in range(_N_CHUNKS):
        pltpu.sync_copy(ones_v, shared.at[idx_v.at[j]], add=True)

    plsc.subcore_barrier()

    pltpu.sync_copy(
        shared.at[pl.ds(sid * _SLICE, _SLICE)],
        out_hbm.at[cid, pl.ds(sid * _SLICE, _SLICE)],
    )


def _tc_loss_body(counts_ref, x_ref, clicked_ref, out_ref):
    c = counts_ref[0] + counts_ref[1]
    row = lax.broadcasted_iota(jnp.int32, (_ROWS, _LANE), 0)
    col = lax.broadcasted_iota(jnp.int32, (_ROWS, _LANE), 1)
    ids = row * _LANE + col
    x = x_ref[...]

    mult = jnp.zeros((_ROWS, _LANE), jnp.float32)
    for i in range(_N_CLICKED):
        mult = mult + jnp.where(ids == clicked_ref[i], 1.0, 0.0)

    valid = ids < _N_NODES
    present = (c > 0.0) & valid
    neg = present & (mult == 0.0)

    sp_p = jnp.maximum(x, 0.0) + jnp.log1p(jnp.exp(-jnp.abs(x)))
    sp_n = sp_p - x

    s_neg = jnp.sum(jnp.where(neg, sp_p, 0.0))
    n_neg = jnp.sum(jnp.where(neg, 1.0, 0.0))
    rp_minus = jnp.sum(mult * sp_p)
    rp_plus = jnp.sum(mult * sp_n)

    loss_p = _PI * rp_plus / _N_CLICKED
    loss_u = jnp.maximum(s_neg / n_neg - _PI * rp_minus / _N_CLICKED, 0.0)
    out_ref[0, 0] = loss_p + loss_u


_tc_loss = pl.pallas_call(
    _tc_loss_body,
    out_shape=jax.ShapeDtypeStruct((1, 1), jnp.float32),
    in_specs=[
        pl.BlockSpec(memory_space=pltpu.VMEM),
        pl.BlockSpec(memory_space=pltpu.VMEM),
        pl.BlockSpec(memory_space=pltpu.SMEM),
    ],
    out_specs=pl.BlockSpec(memory_space=pltpu.SMEM),
)


def kernel(input, nodes, clicked):
    nodes3 = nodes.reshape(2, _NW, _N_CHUNKS, _CHUNK)
    zeros = jnp.zeros((_NPAD,), jnp.float32)
    ones = jnp.ones((_CHUNK,), jnp.float32)
    counts = _sc_count(nodes3, zeros, ones)
    x2 = jnp.pad(input, (0, _NPAD - _N_NODES)).reshape(_ROWS, _LANE)
    c3 = counts.reshape(_NC, _ROWS, _LANE)
    loss = _tc_loss(c3, x2, clicked)
    return loss[0, 0]


# speedup vs baseline: 55.9715x; 1.1872x over previous
import functools

import jax
import jax.numpy as jnp
from jax import lax
from jax.experimental import pallas as pl
from jax.experimental.pallas import tpu as pltpu
from jax.experimental.pallas import tpu_sc as plsc

_N_NODES = 100000
_N_EDGES = 3276800
_N_CLICKED = 64
_PI = 0.25

_LANE = 128
_ROWS = 784
_NPAD = _ROWS * _LANE
_NC, _NS = 2, 16
_NW = _NC * _NS
_ROWS_PER_W = _N_EDGES // _NW // _LANE
_SLICE = _NPAD // _NS

_sc_mesh = plsc.VectorSubcoreMesh(
    core_axis_name="c", subcore_axis_name="s", num_cores=_NC, num_subcores=_NS
)


_N_CHUNKS = 5
_CHUNK = _N_EDGES // _NW // _N_CHUNKS


@functools.partial(
    pl.kernel,
    out_type=jax.ShapeDtypeStruct((_NC, _NPAD), jnp.float32),
    mesh=_sc_mesh,
    scratch_types=[
        pltpu.VMEM((_N_CHUNKS * _CHUNK,), jnp.int32),
        pltpu.VMEM((_CHUNK,), jnp.float32),
        pltpu.VMEM_SHARED((_NPAD,), jnp.float32),
    ],
    compiler_params=pltpu.CompilerParams(use_tc_tiling_on_sc=False),
)
def _sc_count(nodes0_hbm, zeros_hbm, ones_hbm, out_hbm, idx_v, ones_v, shared):
    cid = lax.axis_index("c")
    sid = lax.axis_index("s")

    pltpu.sync_copy(
        zeros_hbm.at[pl.ds(sid * _SLICE, _SLICE)],
        shared.at[pl.ds(sid * _SLICE, _SLICE)],
    )
    pltpu.sync_copy(ones_hbm, ones_v)
    wid = cid * _NS + sid
    pltpu.sync_copy(nodes0_hbm.at[pl.ds(wid * _N_CHUNKS * _CHUNK, _N_CHUNKS * _CHUNK)], idx_v)

    plsc.subcore_barrier()

    for j in range(_N_CHUNKS):
        pltpu.sync_copy(ones_v, shared.at[idx_v.at[pl.ds(j * _CHUNK, _CHUNK)]], add=True)

    plsc.subcore_barrier()

    pltpu.sync_copy(
        shared.at[pl.ds(sid * _SLICE, _SLICE)],
        out_hbm.at[cid, pl.ds(sid * _SLICE, _SLICE)],
    )


def _tc_loss_body(counts_ref, x_ref, clicked_ref, out_ref):
    c = counts_ref[0] + counts_ref[1]
    row = lax.broadcasted_iota(jnp.int32, (_ROWS, _LANE), 0)
    col = lax.broadcasted_iota(jnp.int32, (_ROWS, _LANE), 1)
    ids = row * _LANE + col
    x = x_ref[...]

    mult = jnp.zeros((_ROWS, _LANE), jnp.float32)
    for i in range(_N_CLICKED):
        mult = mult + jnp.where(ids == clicked_ref[i], 1.0, 0.0)

    valid = ids < _N_NODES
    present = (c > 0.0) & valid
    neg = present & (mult == 0.0)

    sp_p = jnp.maximum(x, 0.0) + jnp.log1p(jnp.exp(-jnp.abs(x)))
    sp_n = sp_p - x

    s_neg = jnp.sum(jnp.where(neg, sp_p, 0.0))
    n_neg = jnp.sum(jnp.where(neg, 1.0, 0.0))
    rp_minus = jnp.sum(mult * sp_p)
    rp_plus = jnp.sum(mult * sp_n)

    loss_p = _PI * rp_plus / _N_CLICKED
    loss_u = jnp.maximum(s_neg / n_neg - _PI * rp_minus / _N_CLICKED, 0.0)
    out_ref[0, 0] = loss_p + loss_u


_tc_loss = pl.pallas_call(
    _tc_loss_body,
    out_shape=jax.ShapeDtypeStruct((1, 1), jnp.float32),
    in_specs=[
        pl.BlockSpec(memory_space=pltpu.VMEM),
        pl.BlockSpec(memory_space=pltpu.VMEM),
        pl.BlockSpec(memory_space=pltpu.SMEM),
    ],
    out_specs=pl.BlockSpec(memory_space=pltpu.SMEM),
)


def kernel(input, nodes, clicked):
    nodes0 = jnp.ravel(nodes[0])
    zeros = jnp.zeros((_NPAD,), jnp.float32)
    ones = jnp.ones((_CHUNK,), jnp.float32)
    counts = _sc_count(nodes0, zeros, ones)
    x2 = jnp.pad(input, (0, _NPAD - _N_NODES)).reshape(_ROWS, _LANE)
    c3 = counts.reshape(_NC, _ROWS, _LANE)
    loss = _tc_loss(c3, x2, clicked)
    return loss[0, 0]


# speedup vs baseline: 56.0001x; 1.0005x over previous
import functools

import jax
import jax.numpy as jnp
from jax import lax
from jax.experimental import pallas as pl
from jax.experimental.pallas import tpu as pltpu
from jax.experimental.pallas import tpu_sc as plsc

_N_NODES = 100000
_N_EDGES = 3276800
_N_CLICKED = 64
_PI = 0.25

_LANE = 128
_ROWS = 784
_NPAD = _ROWS * _LANE
_NC, _NS = 2, 16
_NW = _NC * _NS
_ROWS_PER_W = _N_EDGES // _NW // _LANE
_SLICE = _NPAD // _NS

_sc_mesh = plsc.VectorSubcoreMesh(
    core_axis_name="c", subcore_axis_name="s", num_cores=_NC, num_subcores=_NS
)


_N_CHUNKS = 5
_CHUNK = _N_EDGES // _NW // _N_CHUNKS


@functools.partial(
    pl.kernel,
    out_type=jax.ShapeDtypeStruct((_NC, _NPAD), jnp.float32),
    mesh=_sc_mesh,
    scratch_types=[
        pltpu.VMEM((_N_CHUNKS * _CHUNK,), jnp.int32),
        pltpu.VMEM((_CHUNK,), jnp.float32),
        pltpu.VMEM_SHARED((_NPAD,), jnp.float32),
    ],
    compiler_params=pltpu.CompilerParams(use_tc_tiling_on_sc=False),
)
def _sc_count(nodes0_hbm, zeros_hbm, ones_hbm, out_hbm, idx_v, ones_v, shared):
    cid = lax.axis_index("c")
    sid = lax.axis_index("s")

    pltpu.sync_copy(
        zeros_hbm.at[pl.ds(sid * _SLICE, _SLICE)],
        shared.at[pl.ds(sid * _SLICE, _SLICE)],
    )
    pltpu.sync_copy(ones_hbm, ones_v)
    wid = cid * _NS + sid
    pltpu.sync_copy(nodes0_hbm.at[pl.ds(wid * _N_CHUNKS * _CHUNK, _N_CHUNKS * _CHUNK)], idx_v)

    plsc.subcore_barrier()

    for j in range(_N_CHUNKS):
        pltpu.sync_copy(ones_v, shared.at[idx_v.at[pl.ds(j * _CHUNK, _CHUNK)]])

    plsc.subcore_barrier()

    pltpu.sync_copy(
        shared.at[pl.ds(sid * _SLICE, _SLICE)],
        out_hbm.at[cid, pl.ds(sid * _SLICE, _SLICE)],
    )


def _tc_loss_body(counts_ref, x_ref, clicked_ref, out_ref):
    c = counts_ref[0] + counts_ref[1]
    row = lax.broadcasted_iota(jnp.int32, (_ROWS, _LANE), 0)
    col = lax.broadcasted_iota(jnp.int32, (_ROWS, _LANE), 1)
    ids = row * _LANE + col
    x = x_ref[...]

    mult = jnp.zeros((_ROWS, _LANE), jnp.float32)
    for i in range(_N_CLICKED):
        mult = mult + jnp.where(ids == clicked_ref[i], 1.0, 0.0)

    valid = ids < _N_NODES
    present = (c > 0.0) & valid
    neg = present & (mult == 0.0)

    sp_p = jnp.maximum(x, 0.0) + jnp.log1p(jnp.exp(-jnp.abs(x)))
    sp_n = sp_p - x

    s_neg = jnp.sum(jnp.where(neg, sp_p, 0.0))
    n_neg = jnp.sum(jnp.where(neg, 1.0, 0.0))
    rp_minus = jnp.sum(mult * sp_p)
    rp_plus = jnp.sum(mult * sp_n)

    loss_p = _PI * rp_plus / _N_CLICKED
    loss_u = jnp.maximum(s_neg / n_neg - _PI * rp_minus / _N_CLICKED, 0.0)
    out_ref[0, 0] = loss_p + loss_u


_tc_loss = pl.pallas_call(
    _tc_loss_body,
    out_shape=jax.ShapeDtypeStruct((1, 1), jnp.float32),
    in_specs=[
        pl.BlockSpec(memory_space=pltpu.VMEM),
        pl.BlockSpec(memory_space=pltpu.VMEM),
        pl.BlockSpec(memory_space=pltpu.SMEM),
    ],
    out_specs=pl.BlockSpec(memory_space=pltpu.SMEM),
)


def kernel(input, nodes, clicked):
    nodes0 = jnp.ravel(nodes[0])
    zeros = jnp.zeros((_NPAD,), jnp.float32)
    ones = jnp.ones((_CHUNK,), jnp.float32)
    counts = _sc_count(nodes0, zeros, ones)
    x2 = jnp.pad(input, (0, _NPAD - _N_NODES)).reshape(_ROWS, _LANE)
    c3 = counts.reshape(_NC, _ROWS, _LANE)
    loss = _tc_loss(c3, x2, clicked)
    return loss[0, 0]
